# Initial kernel scaffold; baseline (speedup 1.0000x reference)
import jax
import jax.numpy as jnp
from jax.experimental import pallas as pl


def kernel(latents, embedding_weight):
    raise NotImplementedError("write your pallas kernel here")


# speedup vs baseline: 6.7815x; 6.7815x over previous
import jax
import jax.numpy as jnp
from jax.experimental import pallas as pl
from jax.experimental.pallas import tpu as pltpu

_K = 8192
_D = 32
_BETA = 0.25
_TN = 256
_TW = 4096


def _vq_tc_kernel(z_ref, e_ref, q_ref, loss_ref):
    z = z_ref[...]
    e = e_ref[...]
    zb = z.astype(jnp.bfloat16).astype(jnp.float32)
    prod = jax.lax.dot_general(zb, e, (((1,), (1,)), ((), ())),
                               preferred_element_type=jnp.float32)
    z2 = jnp.sum(z * z, axis=1, keepdims=True)
    e2 = jnp.sum(e * e, axis=1)
    dist = (z2 + e2[None, :]) - 2.0 * prod

    acc_v = None
    acc_i = None
    for t in range(_K // _TW):
        dt = dist[:, t * _TW:(t + 1) * _TW]
        mval = jnp.min(dt, axis=1, keepdims=True)
        ii = jax.lax.broadcasted_iota(jnp.int32, dt.shape, 1) + t * _TW
        lidx = jnp.min(jnp.where(dt <= mval, ii, _K), axis=1)
        xv = mval[:, 0]
        xvb = xv.astype(jnp.bfloat16).astype(jnp.float32)
        if acc_v is None:
            acc_v, acc_i = xvb, lidx
        else:
            keep = (acc_v < xv) | ((acc_v == xv) & (acc_i < lidx))
            acc_v = jnp.where(keep, acc_v, xvb)
            acc_i = jnp.where(keep, acc_i, lidx)

    kk = jax.lax.broadcasted_iota(jnp.int32, (_TN, _K), 1)
    onehot = (acc_i[:, None] == kk).astype(jnp.float32)
    q = jax.lax.dot_general(onehot, e, (((1,), (0,)), ((), ())),
                            preferred_element_type=jnp.float32)
    diff = q - z
    q_ref[...] = z + diff

    @pl.when(pl.program_id(0) == 0)
    def _init():
        loss_ref[...] = jnp.zeros((1, 1), jnp.float32)

    loss_ref[...] += jnp.reshape(jnp.sum(diff * diff), (1, 1))


def kernel(latents, embedding_weight):
    latents_shape = latents.shape
    flat = latents.reshape(-1, _D)
    n = flat.shape[0]
    grid = n // _TN
    q, loss_sum = pl.pallas_call(
        _vq_tc_kernel,
        grid=(grid,),
        in_specs=[
            pl.BlockSpec((_TN, _D), lambda i: (i, 0)),
            pl.BlockSpec((_K, _D), lambda i: (0, 0)),
        ],
        out_specs=[
            pl.BlockSpec((_TN, _D), lambda i: (i, 0)),
            pl.BlockSpec((1, 1), lambda i: (0, 0)),
        ],
        out_shape=[
            jax.ShapeDtypeStruct((n, _D), jnp.float32),
            jax.ShapeDtypeStruct((1, 1), jnp.float32),
        ],
        compiler_params=pltpu.CompilerParams(
            dimension_semantics=("arbitrary",),
        ),
    )(flat, embedding_weight)
    mse = loss_sum[0, 0] / (n * _D)
    vq_loss = mse * _BETA + mse
    return q.reshape(latents_shape), vq_loss
